# baseline (speedup 1.0000x reference)
import jax
import jax.numpy as jnp
from jax import lax
from jax.experimental import pallas as pl
from jax.experimental.pallas import tpu as pltpu

_TOP_K = 96


def _pick_spatial_chunk(h: int) -> int:
    for cand in (32, 16, 8):
        if h % cand == 0:
            return cand
    return h


def _score_body(x_ref, s_ref):
    part = jnp.max(jnp.abs(x_ref[...]), axis=(2, 3))
    part3 = part[:, None, :]

    @pl.when(pl.program_id(1) == 0)
    def _init():
        s_ref[...] = part3

    @pl.when(pl.program_id(1) != 0)
    def _acc():
        s_ref[...] = jnp.maximum(s_ref[...], part3)


def _topk_body(k: int, s_ref, o_ref):
    s = s_ref[:, 0, :]
    b, c = s.shape
    si = s[:, :, None]
    sj = s[:, None, :]
    ii = lax.broadcasted_iota(jnp.int32, (b, c, c), 1)
    jj = lax.broadcasted_iota(jnp.int32, (b, c, c), 2)
    beats = (sj > si) | ((sj == si) & (jj < ii))
    rank = jnp.sum(beats.astype(jnp.int32), axis=2)
    pos = lax.broadcasted_iota(jnp.int32, (b, c, k), 2)
    chan = lax.broadcasted_iota(jnp.int32, (b, c, k), 1)
    hit = rank[:, :, None] == pos
    idx = jnp.sum(jnp.where(hit, chan, 0), axis=1)
    o_ref[...] = idx[:, None, :]


def _gather_body(k: int, window: int, idx_ref, x_ref, o_ref, sem):
    n = o_ref.shape[0] * k

    def issue(i, carry):
        b = i // k
        j = i - b * k
        ch = idx_ref[b, 0, j]
        pltpu.make_async_copy(x_ref.at[b, ch], o_ref.at[b, j], sem).start()

        @pl.when(i >= window)
        def _():
            pltpu.make_async_copy(x_ref.at[0, 0], o_ref.at[0, 0], sem).wait()

        return carry

    lax.fori_loop(0, n, issue, 0)

    def drain(i, carry):
        pltpu.make_async_copy(x_ref.at[0, 0], o_ref.at[0, 0], sem).wait()
        return carry

    lax.fori_loop(0, min(window, n), drain, 0)


def _channel_topk_pool(x, k: int):
    b, c, h, w = x.shape
    hc = _pick_spatial_chunk(h)
    nsp = h // hc

    scores = pl.pallas_call(
        _score_body,
        grid=(b, nsp),
        in_specs=[pl.BlockSpec((1, c, hc, w), lambda bi, si: (bi, 0, si, 0))],
        out_specs=pl.BlockSpec((1, 1, c), lambda bi, si: (bi, 0, 0)),
        out_shape=jax.ShapeDtypeStruct((b, 1, c), jnp.float32),
    )(x)

    idx = pl.pallas_call(
        lambda s_ref, o_ref: _topk_body(k, s_ref, o_ref),
        in_specs=[pl.BlockSpec((b, 1, c), lambda: (0, 0, 0))],
        out_specs=pl.BlockSpec((b, 1, k), lambda: (0, 0, 0)),
        out_shape=jax.ShapeDtypeStruct((b, 1, k), jnp.int32),
    )(scores)

    out = pl.pallas_call(
        lambda idx_ref, x_ref, o_ref, sem: _gather_body(
            k, 64, idx_ref, x_ref, o_ref, sem
        ),
        in_specs=[
            pl.BlockSpec(memory_space=pltpu.SMEM),
            pl.BlockSpec(memory_space=pl.ANY),
        ],
        out_specs=pl.BlockSpec(memory_space=pl.ANY),
        out_shape=jax.ShapeDtypeStruct((b, k, h, w), jnp.float32),
        scratch_shapes=[pltpu.SemaphoreType.DMA],
    )(idx, x)
    return out


def kernel(x):
    return _channel_topk_pool(x, _TOP_K)


# speedup vs baseline: 3.0941x; 3.0941x over previous
import jax
import jax.numpy as jnp
from jax import lax
from jax.experimental import pallas as pl
from jax.experimental.pallas import tpu as pltpu

_TOP_K = 96


def _score_body(x_ref, o_ref, acc_ref):
    c = pl.program_id(1)
    nc = pl.num_programs(1)
    a = jnp.abs(x_ref[...])

    @pl.when(c == 0)
    def _init():
        acc_ref[...] = a

    @pl.when(c != 0)
    def _acc():
        acc_ref[...] = jnp.maximum(acc_ref[...], a)

    @pl.when(c == nc - 1)
    def _fin():
        o_ref[...] = jnp.max(acc_ref[...], axis=1, keepdims=True)


def _topk_body(k: int, s_ref, o_ref):
    s = s_ref[...]
    b, c = s.shape
    si = s[:, :, None]
    sj = s[:, None, :]
    ii = lax.broadcasted_iota(jnp.int32, (b, c, c), 1)
    jj = lax.broadcasted_iota(jnp.int32, (b, c, c), 2)
    beats = (sj > si) | ((sj == si) & (jj < ii))
    rank = jnp.sum(beats.astype(jnp.int32), axis=2)
    pos = lax.broadcasted_iota(jnp.int32, (b, c, k), 2)
    chan = lax.broadcasted_iota(jnp.int32, (b, c, k), 1)
    hit = rank[:, :, None] == pos
    o_ref[...] = jnp.sum(jnp.where(hit, chan, 0), axis=1)


def _gather_body(idx_ref, x_ref, o_ref):
    del idx_ref
    o_ref[...] = x_ref[...]


def _channel_topk_pool(x, k: int):
    b, c, h, w = x.shape
    hw = h * w
    rows = b * c
    x2 = x.reshape(rows, hw)

    row_blk = 256
    col_blk = 3584
    grid_r = rows // row_blk
    grid_c = hw // col_blk

    scores2 = pl.pallas_call(
        _score_body,
        grid=(grid_r, grid_c),
        in_specs=[pl.BlockSpec((row_blk, col_blk), lambda r, cc: (r, cc))],
        out_specs=pl.BlockSpec((row_blk, 1), lambda r, cc: (r, 0)),
        out_shape=jax.ShapeDtypeStruct((rows, 1), jnp.float32),
        scratch_shapes=[pltpu.VMEM((row_blk, col_blk), jnp.float32)],
    )(x2)
    scores = scores2.reshape(b, c)

    idx = pl.pallas_call(
        lambda s_ref, o_ref: _topk_body(k, s_ref, o_ref),
        in_specs=[pl.BlockSpec((b, c), lambda: (0, 0))],
        out_specs=pl.BlockSpec((b, k), lambda: (0, 0)),
        out_shape=jax.ShapeDtypeStruct((b, k), jnp.int32),
    )(scores)

    out = pl.pallas_call(
        _gather_body,
        grid_spec=pltpu.PrefetchScalarGridSpec(
            num_scalar_prefetch=1,
            grid=(b, k),
            in_specs=[
                pl.BlockSpec(
                    (1, 1, h, w), lambda bi, ki, idx_r: (bi, idx_r[bi, ki], 0, 0)
                )
            ],
            out_specs=pl.BlockSpec((1, 1, h, w), lambda bi, ki, idx_r: (bi, ki, 0, 0)),
        ),
        out_shape=jax.ShapeDtypeStruct((b, k, h, w), jnp.float32),
    )(idx, x)
    return out


def kernel(x):
    return _channel_topk_pool(x, _TOP_K)


# speedup vs baseline: 3.7432x; 1.2098x over previous
import functools

import jax
import jax.numpy as jnp
from jax import lax
from jax.experimental import pallas as pl
from jax.experimental.pallas import tpu as pltpu
from jax.experimental.pallas import tpu_sc as plsc

_TOP_K = 96
_LANES = 16
_NUM_WORKERS = 32
_UNROLL = 16


def _reduce_row(buf, hw, acc_ref, row_local):
    chunk = _UNROLL * _LANES
    n_outer = hw // chunk

    def body(j, acc):
        base = j * chunk
        vs = [buf[pl.ds(base + u * _LANES, _LANES)] for u in range(_UNROLL)]
        m = [jnp.abs(v) for v in vs]
        while len(m) > 1:
            m = [jnp.maximum(m[i], m[i + 1]) for i in range(0, len(m) - 1, 2)] + (
                [m[-1]] if len(m) % 2 else []
            )
        return jnp.maximum(acc, m[0])

    acc = lax.fori_loop(0, n_outer, body, jnp.zeros((_LANES,), jnp.float32))
    acc_ref[pl.ds(row_local * _LANES, _LANES)] = acc


def _sc_score_kernel(rows, hw):
    rpw = rows // _NUM_WORKERS
    mesh = plsc.VectorSubcoreMesh(core_axis_name="c", subcore_axis_name="s")

    @functools.partial(
        pl.kernel,
        mesh=mesh,
        out_type=jax.ShapeDtypeStruct((rows * _LANES,), jnp.float32),
        scratch_types=[
            pltpu.VMEM((hw,), jnp.float32),
            pltpu.VMEM((hw,), jnp.float32),
            pltpu.VMEM((rpw * _LANES,), jnp.float32),
            pltpu.SemaphoreType.DMA,
            pltpu.SemaphoreType.DMA,
        ],
    )
    def k(x_hbm, out_hbm, buf0, buf1, acc, sem0, sem1):
        wid = lax.axis_index("s") * 2 + lax.axis_index("c")
        base = wid * rpw

        pltpu.make_async_copy(x_hbm.at[base], buf0, sem0).start()

        def outer(i, _):
            r0 = base + i * 2
            pltpu.make_async_copy(x_hbm.at[r0 + 1], buf1, sem1).start()
            pltpu.make_async_copy(x_hbm.at[r0], buf0, sem0).wait()
            _reduce_row(buf0, hw, acc, i * 2)

            @pl.when(i * 2 + 2 < rpw)
            def _():
                pltpu.make_async_copy(x_hbm.at[r0 + 2], buf0, sem0).start()

            pltpu.make_async_copy(x_hbm.at[r0 + 1], buf1, sem1).wait()
            _reduce_row(buf1, hw, acc, i * 2 + 1)
            return 0

        lax.fori_loop(0, rpw // 2, outer, 0)
        pltpu.sync_copy(acc, out_hbm.at[pl.ds(base * _LANES, rpw * _LANES)])

    return k


def _topk_body(k: int, s_ref, o_ref, s2_ref):
    s2_ref[...] = jnp.max(s_ref[...], axis=2)
    s = s2_ref[...]
    b, c = s.shape
    si = s[:, :, None]
    sj = s[:, None, :]
    ii = lax.broadcasted_iota(jnp.int32, (b, c, c), 1)
    jj = lax.broadcasted_iota(jnp.int32, (b, c, c), 2)
    beats = (sj > si) | ((sj == si) & (jj < ii))
    rank = jnp.sum(beats.astype(jnp.int32), axis=2)
    pos = lax.broadcasted_iota(jnp.int32, (b, c, k), 2)
    chan = lax.broadcasted_iota(jnp.int32, (b, c, k), 1)
    hit = rank[:, :, None] == pos
    o_ref[...] = jnp.sum(jnp.where(hit, chan, 0), axis=1)


_GATHER_CHUNK = 16


def _gather_body(idx_ref, *refs):
    del idx_ref
    xs = refs[:-1]
    o_ref = refs[-1]
    for j, x_ref in enumerate(xs):
        o_ref[0, j] = x_ref[0, 0]


def _channel_topk_pool(x, k: int):
    b, c, h, w = x.shape
    hw = h * w
    rows = b * c
    x2 = x.reshape(rows, hw)

    scores16 = _sc_score_kernel(rows, hw)(x2)
    s3 = scores16.reshape(b, c, _LANES)

    idx = pl.pallas_call(
        functools.partial(_topk_body, k),
        in_specs=[pl.BlockSpec((b, c, _LANES), lambda: (0, 0, 0))],
        out_specs=pl.BlockSpec((b, k), lambda: (0, 0)),
        out_shape=jax.ShapeDtypeStruct((b, k), jnp.int32),
        scratch_shapes=[pltpu.VMEM((b, c), jnp.float32)],
    )(s3)

    g = _GATHER_CHUNK

    def _in_spec(j):
        return pl.BlockSpec(
            (1, 1, h, w), lambda bi, ki, idx_r: (bi, idx_r[bi, ki * g + j], 0, 0)
        )

    out = pl.pallas_call(
        _gather_body,
        grid_spec=pltpu.PrefetchScalarGridSpec(
            num_scalar_prefetch=1,
            grid=(b, k // g),
            in_specs=[_in_spec(j) for j in range(g)],
            out_specs=pl.BlockSpec((1, g, h, w), lambda bi, ki, idx_r: (bi, ki, 0, 0)),
        ),
        out_shape=jax.ShapeDtypeStruct((b, k, h, w), jnp.float32),
    )(idx, *([x] * g))
    return out


def kernel(x):
    return _channel_topk_pool(x, _TOP_K)
